# baseline (speedup 1.0000x reference)
import jax
import jax.numpy as jnp
from jax import lax
from jax.experimental import pallas as pl
from jax.experimental.pallas import tpu as pltpu
from jax.experimental.pallas import tpu_sc as plsc

_T = 8192
_BC = 8
_NQ = 4
_TCH = _T // _NQ
_OUT = _T * 8 + 8
_OUTPAD = _T * 8 + 128
_CONTRIB = _TCH * 8 + 128
_PC = 128 + _TCH + 16


def _pool_body(x_ref, o_ref):
    x = x_ref[0]
    o_ref[0] = x.reshape(
        x.shape[0] // 16, 16, x.shape[-1]).sum(axis=1) * (1.0 / 16.0)


def _oa_body(pooled, out, pbuf, contrib):
    c = lax.axis_index("c")
    s = lax.axis_index("s")
    w = c * 16 + s
    bc = w // _NQ
    q = w % _NQ
    t0 = q * _TCH

    zero16 = jnp.zeros((16,), jnp.float32)
    lane = lax.iota(jnp.int32, 16)
    rlow = lane & 7
    rhigh = rlow + 8
    colpat = lane >> 3

    for j in range(16):
        pbuf[j, pl.ds(112, 16)] = zero16
        pbuf[j, pl.ds(128 + _TCH, 16)] = zero16

    @pl.when(q > 0)
    def _():
        pltpu.sync_copy(pooled.at[bc, :, pl.ds(t0 - 128, 128 + _TCH)],
                        pbuf.at[:, pl.ds(0, 128 + _TCH)])

    @pl.when(q == 0)
    def _():
        pltpu.sync_copy(pooled.at[bc, :, pl.ds(0, _TCH)],
                        pbuf.at[:, pl.ds(128, _TCH)])

    @plsc.parallel_loop(0, _TCH // 2 + 1, unroll=8)
    def pair_body(h):
        cola = colpat + (128 + 2 * h)
        ta = plsc.load_gather(pbuf, [rlow, cola])
        tb = plsc.load_gather(pbuf, [rhigh, cola - 1])
        contrib[pl.ds(h * 16, 16)] = ta + tb

    pltpu.sync_copy(contrib.at[pl.ds(0, _TCH * 8)],
                    out.at[bc, pl.ds(t0 * 8, _TCH * 8)])

    @pl.when(q == _NQ - 1)
    def _():
        pltpu.sync_copy(contrib.at[pl.ds(_TCH * 8, 128)],
                        out.at[bc, pl.ds(_T * 8, 128)])


def kernel(est_source):
    est = est_source.reshape(_BC, 256, _T)

    pooled = pl.pallas_call(
        _pool_body,
        out_shape=jax.ShapeDtypeStruct((_BC, 16, _T), jnp.float32),
        grid=(_BC * 2,),
        in_specs=[pl.BlockSpec((1, 128, _T), lambda i: (i // 2, i % 2, 0))],
        out_specs=pl.BlockSpec((1, 8, _T), lambda i: (i // 2, i % 2, 0)),
        compiler_params=pltpu.CompilerParams(
            dimension_semantics=("parallel",)),
    )(est)

    mesh = plsc.VectorSubcoreMesh(core_axis_name="c", subcore_axis_name="s")
    out = pl.kernel(
        _oa_body,
        out_type=jax.ShapeDtypeStruct((_BC, _OUTPAD), jnp.float32),
        mesh=mesh,
        scratch_types=[
            pltpu.VMEM((16, _PC), jnp.float32),
            pltpu.VMEM((_CONTRIB,), jnp.float32),
        ],
        compiler_params=pltpu.CompilerParams(needs_layout_passes=False),
    )(pooled)
    return out[:, :_OUT].reshape(4, 2, _OUT)


# speedup vs baseline: 1.0536x; 1.0536x over previous
import jax
import jax.numpy as jnp
from jax import lax
from jax.experimental import pallas as pl
from jax.experimental.pallas import tpu as pltpu
from jax.experimental.pallas import tpu_sc as plsc

_T = 8192
_BC = 8
_NQ = 4
_TCH = _T // _NQ
_OUT = _T * 8 + 8
_OUTPAD = _T * 8 + 128
_CONTRIB = _TCH * 8 + 128
_PC = 128 + _TCH + 16


def _pool_body(x_ref, o_ref):
    x = x_ref[0]
    o_ref[0] = x.reshape(16, 16, x.shape[-1]).sum(axis=1) * (1.0 / 16.0)


def _oa_body(pooled, out, pbuf, contrib, sem1, sem2, sem3):
    c = lax.axis_index("c")
    s = lax.axis_index("s")
    w = c * 16 + s
    bc = w // _NQ
    q = w % _NQ
    t0 = q * _TCH

    zero16 = jnp.zeros((16,), jnp.float32)
    lane = lax.iota(jnp.int32, 16)
    rlow = lane & 7
    rhigh = rlow + 8
    colpat = lane >> 3

    for j in range(16):
        pbuf[j, pl.ds(112, 16)] = zero16
        pbuf[j, pl.ds(128 + _TCH, 16)] = zero16

    half = _TCH // 2
    @pl.when(q > 0)
    def _():
        pltpu.async_copy(pooled.at[bc, :, pl.ds(t0 - 128, 128 + half)],
                         pbuf.at[:, pl.ds(0, 128 + half)], sem1)

    @pl.when(q == 0)
    def _():
        pltpu.async_copy(pooled.at[bc, :, pl.ds(0, half)],
                         pbuf.at[:, pl.ds(128, half)], sem1)

    cp2 = pltpu.async_copy(pooled.at[bc, :, pl.ds(t0 + half, half)],
                           pbuf.at[:, pl.ds(128 + half, half)], sem2)

    @pl.when(q > 0)
    def _():
        pltpu.make_async_copy(pooled.at[bc, :, pl.ds(t0 - 128, 128 + half)],
                              pbuf.at[:, pl.ds(0, 128 + half)], sem1).wait()

    @pl.when(q == 0)
    def _():
        pltpu.make_async_copy(pooled.at[bc, :, pl.ds(0, half)],
                              pbuf.at[:, pl.ds(128, half)], sem1).wait()

    @plsc.parallel_loop(0, half // 2, unroll=8)
    def pair_body(h):
        cola = colpat + (128 + 2 * h)
        ta = plsc.load_gather(pbuf, [rlow, cola])
        tb = plsc.load_gather(pbuf, [rhigh, cola - 1])
        contrib[pl.ds(h * 16, 16)] = ta + tb

    cp_out1 = pltpu.async_copy(contrib.at[pl.ds(0, half * 8)],
                               out.at[bc, pl.ds(t0 * 8, half * 8)], sem3)
    cp2.wait()

    @plsc.parallel_loop(half // 2, _TCH // 2 + 1, unroll=8)
    def pair_body2(h):
        cola = colpat + (128 + 2 * h)
        ta = plsc.load_gather(pbuf, [rlow, cola])
        tb = plsc.load_gather(pbuf, [rhigh, cola - 1])
        contrib[pl.ds(h * 16, 16)] = ta + tb

    pltpu.sync_copy(contrib.at[pl.ds(half * 8, half * 8)],
                    out.at[bc, pl.ds(t0 * 8 + half * 8, half * 8)])

    @pl.when(q == _NQ - 1)
    def _():
        pltpu.sync_copy(contrib.at[pl.ds(_TCH * 8, 128)],
                        out.at[bc, pl.ds(_T * 8, 128)])

    cp_out1.wait()


def kernel(est_source):
    est = est_source.reshape(_BC, 256, _T)

    pooled = pl.pallas_call(
        _pool_body,
        out_shape=jax.ShapeDtypeStruct((_BC, 16, _T), jnp.float32),
        grid=(_BC,),
        in_specs=[pl.BlockSpec((1, 256, _T), lambda b: (b, 0, 0))],
        out_specs=pl.BlockSpec((1, 16, _T), lambda b: (b, 0, 0)),
        compiler_params=pltpu.CompilerParams(
            dimension_semantics=("parallel",)),
    )(est)

    mesh = plsc.VectorSubcoreMesh(core_axis_name="c", subcore_axis_name="s")
    out = pl.kernel(
        _oa_body,
        out_type=jax.ShapeDtypeStruct((_BC, _OUTPAD), jnp.float32),
        mesh=mesh,
        scratch_types=[
            pltpu.VMEM((16, _PC), jnp.float32),
            pltpu.VMEM((_CONTRIB,), jnp.float32),
            pltpu.SemaphoreType.DMA,
            pltpu.SemaphoreType.DMA,
            pltpu.SemaphoreType.DMA,
        ],
        compiler_params=pltpu.CompilerParams(needs_layout_passes=False),
    )(pooled)
    return out[:, :_OUT].reshape(4, 2, _OUT)
